# baseline (speedup 1.0000x reference)
import functools
import jax
import jax.numpy as jnp
from jax import lax
from jax.experimental import pallas as pl
from jax.experimental.pallas import tpu as pltpu
from jax.experimental.pallas import tpu_sc as plsc

_NEG = -100000.0
_G = 128


def _sc_pos_amino(T, B, D):
    NW = 32
    CH = B // NW
    CAP = CH * T + 2 * _G
    NGRP = CH // 16
    mesh = plsc.VectorSubcoreMesh(core_axis_name="c", subcore_axis_name="s")

    @functools.partial(
        pl.kernel, mesh=mesh,
        compiler_params=pltpu.CompilerParams(needs_layout_passes=False),
        out_type=(
            jax.ShapeDtypeStruct((B * T,), jnp.float32),
            jax.ShapeDtypeStruct((B, D), jnp.float32),
        ),
        scratch_types=[
            pltpu.VMEM((CH + 16,), jnp.int32),
            pltpu.VMEM((CH,), jnp.int32),
            pltpu.VMEM((CH + 16,), jnp.int32),
            pltpu.VMEM((CAP,), jnp.int32),
            pltpu.VMEM((CH,), jnp.int32),
            pltpu.VMEM((D,), jnp.float32),
            pltpu.VMEM((16,), jnp.float32),
            pltpu.VMEM((2, _G, D), jnp.float32),
            pltpu.VMEM((CH * T,), jnp.float32),
            pltpu.SemaphoreType.DMA,
            pltpu.SemaphoreType.DMA,
        ],
    )
    def k(lat_hbm, len_hbm, posac_hbm, wbc_hbm, bpos_hbm,
          pos_out_hbm, am_out_hbm,
          len_v, posac_v, pref_v, idx_v, amidx_v, wbc_v, bpos_v,
          rows_v, out_v, sem_a, sem_b):
        wid = lax.axis_index("s") * 2 + lax.axis_index("c")
        base = wid * CH
        iota = lax.iota(jnp.int32, 16)

        pltpu.sync_copy(len_hbm.at[pl.ds(base, CH)], len_v.at[pl.ds(0, CH)])
        pltpu.sync_copy(posac_hbm.at[pl.ds(base, CH)], posac_v)
        pltpu.sync_copy(wbc_hbm, wbc_v)
        pltpu.sync_copy(bpos_hbm, bpos_v)

        negv = jnp.full((16,), _NEG, jnp.float32)

        @plsc.parallel_loop(0, CH * T // 16, unroll=8)
        def _(i):
            out_v[pl.ds(i * 16, 16)] = negv

        def perb(l, p):
            ln = len_v[pl.ds(l, 16)][0]
            bg = base + l
            idx_v[pl.ds(p, 16)] = iota * B + bg
            idx_v[pl.ds(p + 16, 16)] = (iota + 16) * B + bg
            return p + ln
        n_rows = lax.fori_loop(0, CH, perb, 0)

        nc = (n_rows + (_G - 1)) // _G
        sentinel = jnp.full((16,), base, jnp.int32)

        def ztail(j, _):
            idx_v[pl.ds(n_rows + j * 16, 16)] = sentinel
            return 0
        lax.fori_loop(0, _G // 16, ztail, 0)

        bposv = bpos_v[...]

        def fire(c, k_buf, sem):
            src = lat_hbm.at[idx_v.at[pl.ds(c * _G, _G)]]
            pltpu.async_copy(src, rows_v.at[k_buf], sem)

        def wait(k_buf, sem):
            pltpu.make_async_copy(lat_hbm.at[pl.ds(0, _G)],
                                  rows_v.at[k_buf], sem).wait()

        bpos_s = bposv[0]
        ws = [wbc_v[pl.ds(i * 16, 16)] for i in range(D // 16)]
        lane0 = iota == 0

        def compute(c, k_buf):
            rows = rows_v.at[k_buf]

            @plsc.parallel_loop(0, _G, unroll=4)
            def _(r):
                rr = rows.at[r]
                part = ws[0] * rr[pl.ds(0, 16)]
                for i in range(1, D // 16):
                    part = part + ws[i] * rr[pl.ds(i * 16, 16)]
                s = jnp.sum(part) + bpos_s
                iv = idx_v[pl.ds(c * _G + r, 16)][0]
                t = lax.shift_right_logical(iv, 14)
                bl = (iv & (B - 1)) - base
                posn = bl * T + t
                plsc.store_scatter(out_v, [jnp.full((16,), posn, jnp.int32)],
                                   jnp.full((16,), s, jnp.float32), mask=lane0)

        def chunk(c, _):
            @pl.when(c % 2 == 0)
            def _():
                @pl.when(c + 1 < nc)
                def _():
                    fire(c + 1, 1, sem_b)
                wait(0, sem_a)
                compute(c, 0)

            @pl.when(c % 2 == 1)
            def _():
                @pl.when(c + 1 < nc)
                def _():
                    fire(c + 1, 0, sem_a)
                wait(1, sem_b)
                compute(c, 1)
            return 0

        @pl.when(nc > 0)
        def _():
            fire(0, 0, sem_a)
        lax.fori_loop(0, nc, chunk, 0)

        pltpu.sync_copy(out_v, pos_out_hbm.at[pl.ds(base * T, CH * T)])

        def amb(g, _):
            pv = posac_v[pl.ds(g * 16, 16)]
            amidx_v[pl.ds(g * 16, 16)] = pv * B + (base + g * 16) + iota
            return 0
        lax.fori_loop(0, NGRP, amb, 0)
        for q in range(CH // _G):
            src = lat_hbm.at[amidx_v.at[pl.ds(q * _G, _G)]]
            pltpu.async_copy(src, rows_v.at[0], sem_a).wait()
            pltpu.sync_copy(rows_v.at[0],
                            am_out_hbm.at[pl.ds(base + q * _G, _G)])

    return k


def _am_body(feat_ref, pep_ref, pos_ref, wam_ref, bam_ref, out_ref):
    am = jax.lax.dot_general(feat_ref[...], wam_ref[...],
                             (((1,), (1,)), ((), ())),
                             preferred_element_type=jnp.float32)
    am = am + bam_ref[...]
    pos_ac = pos_ref[...]
    pep = pep_ref[...]
    lane_t = jax.lax.broadcasted_iota(jnp.int32, pep.shape, 1)
    pep_sel = jnp.sum(jnp.where(lane_t == pos_ac, pep, 0), axis=1,
                      keepdims=True)
    mask_col = jnp.where(pep_sel == 0, 19, pep_sel - 1)
    k_iota = jax.lax.broadcasted_iota(jnp.int32, am.shape, 1)
    out_ref[...] = jnp.where(k_iota == mask_col, _NEG, am)


def kernel(latent_amino, latent_pep, peptides, alleles, lengths, pretrain,
           actions, W_pos, b_pos, W_amino, b_amino):
    T, B, D = latent_amino.shape
    lat2d = latent_amino.reshape(T * B, D)
    lens = lengths.astype(jnp.int32)
    pos_ac = actions[:, 0].astype(jnp.int32)
    wbc = W_pos.reshape(-1).astype(jnp.float32)
    bposb = jnp.broadcast_to(b_pos.reshape(1), (16,)).astype(jnp.float32)

    pos_flat, amino_feat = _sc_pos_amino(T, B, D)(
        lat2d, lens, pos_ac, wbc, bposb)
    pos_pd = pos_flat.reshape(B, T)

    BB = 2048
    am = pl.pallas_call(
        _am_body,
        grid=(B // BB,),
        in_specs=[
            pl.BlockSpec((BB, D), lambda i: (i, 0)),
            pl.BlockSpec((BB, T), lambda i: (i, 0)),
            pl.BlockSpec((BB, 1), lambda i: (i, 0)),
            pl.BlockSpec((20, D), lambda i: (0, 0)),
            pl.BlockSpec((1, 20), lambda i: (0, 0)),
        ],
        out_specs=pl.BlockSpec((BB, 20), lambda i: (i, 0)),
        out_shape=jax.ShapeDtypeStruct((B, 20), jnp.float32),
    )(amino_feat, peptides.astype(jnp.int32), pos_ac.reshape(B, 1),
      W_amino, b_amino.reshape(1, -1).astype(jnp.float32))
    return (pos_pd, am)


# speedup vs baseline: 1.5532x; 1.5532x over previous
import jax
import jax.numpy as jnp
from jax.experimental import pallas as pl

_NEG = -100000.0


def _body(lat_ref, len_ref, pos_ref, pep_ref, wpos_ref, bpos_ref,
          wam_ref, bam_ref, out_pos_ref, out_am_ref):
    T, BB, D = lat_ref.shape
    w_row = wpos_ref[...]
    pos_ac = pos_ref[...]
    lens_row = len_ref[...]
    rows = []
    for t in range(T):
        x_t = lat_ref[t]
        s_t = jax.lax.dot_general(w_row, x_t, (((1,), (1,)), ((), ())),
                                  preferred_element_type=jnp.float32)
        rows.append(s_t)
    bits = [pos_ac & (1 << k) > 0 for k in range(5)]
    level = [lat_ref[t] for t in range(T)]
    for k in range(5):
        nxt = []
        for i in range(0, len(level) - 1, 2):
            nxt.append(jnp.where(bits[k], level[i + 1], level[i]))
        if len(level) % 2 == 1:
            nxt.append(level[-1])
        level = nxt
        if len(level) == 1:
            break
    acc = level[0]
    scores_T = jnp.concatenate(rows, axis=0) + bpos_ref[0, 0]
    ti = jax.lax.broadcasted_iota(jnp.int32, (T, BB), 0)
    out_pos_ref[...] = jnp.where(ti < lens_row, scores_T, _NEG)

    am = jax.lax.dot_general(acc, wam_ref[...], (((1,), (1,)), ((), ())),
                             preferred_element_type=jnp.float32)
    am = am + bam_ref[...]
    pep = pep_ref[...]
    lane_t = jax.lax.broadcasted_iota(jnp.int32, (BB, T), 1)
    pep_sel = jnp.sum(jnp.where(lane_t == pos_ac, pep, 0), axis=1,
                      keepdims=True)
    mask_col = jnp.where(pep_sel == 0, 19, pep_sel - 1)
    k_iota = jax.lax.broadcasted_iota(jnp.int32, (BB, 20), 1)
    out_am_ref[...] = jnp.where(k_iota == mask_col, _NEG, am)


def kernel(latent_amino, latent_pep, peptides, alleles, lengths, pretrain,
           actions, W_pos, b_pos, W_amino, b_amino):
    T, B, D = latent_amino.shape
    BB = 1024
    lengths2 = lengths.astype(jnp.int32).reshape(1, B)
    pos_ac = actions[:, 0:1].astype(jnp.int32)
    pep = peptides.astype(jnp.int32)
    bpos2 = b_pos.reshape(1, 1).astype(jnp.float32)
    bam2 = b_amino.reshape(1, -1).astype(jnp.float32)
    f = pl.pallas_call(
        _body,
        grid=(B // BB,),
        in_specs=[
            pl.BlockSpec((T, BB, D), lambda i: (0, i, 0)),
            pl.BlockSpec((1, BB), lambda i: (0, i)),
            pl.BlockSpec((BB, 1), lambda i: (i, 0)),
            pl.BlockSpec((BB, T), lambda i: (i, 0)),
            pl.BlockSpec((1, D), lambda i: (0, 0)),
            pl.BlockSpec((1, 1), lambda i: (0, 0)),
            pl.BlockSpec((20, D), lambda i: (0, 0)),
            pl.BlockSpec((1, 20), lambda i: (0, 0)),
        ],
        out_specs=(
            pl.BlockSpec((T, BB), lambda i: (0, i)),
            pl.BlockSpec((BB, 20), lambda i: (i, 0)),
        ),
        out_shape=(
            jax.ShapeDtypeStruct((T, B), jnp.float32),
            jax.ShapeDtypeStruct((B, 20), jnp.float32),
        ),
    )
    scores_T, amino_pd = f(latent_amino, lengths2, pos_ac, pep, W_pos,
                           bpos2, W_amino, bam2)
    return (scores_T.T, amino_pd)
